# Initial kernel scaffold; baseline (speedup 1.0000x reference)
import functools

import jax
import jax.numpy as jnp
from jax import lax
from jax.experimental import pallas as pl
from jax.experimental.pallas import tpu as pltpu
from jax.experimental.pallas import tpu_sc as plsc

N_NODES = 100000
NSEG = 64
HID = 128
OUT = 128

NC = 2
NS = 16
NW = NC * NS

BLK = 128
NFULL = N_NODES // BLK
NBLK = NFULL + 1
LAST_BASE = N_NODES - BLK
TAIL_PAD = NBLK * BLK - N_NODES


def _sc_segment_sums(x, idxmat):
    mesh = plsc.VectorSubcoreMesh(core_axis_name="c", subcore_axis_name="s")

    @functools.partial(
        pl.kernel,
        out_type=(
            jax.ShapeDtypeStruct((NW, NSEG, HID), jnp.float32),
            jax.ShapeDtypeStruct((NW, NSEG), jnp.float32),
        ),
        mesh=mesh,
        scratch_types=[
            pltpu.VMEM((BLK,), jnp.int32),
            pltpu.VMEM((BLK, HID), jnp.float32),
            pltpu.VMEM((NSEG + 1, HID), jnp.float32),
            pltpu.VMEM((NSEG + 16,), jnp.float32),
        ],
    )
    def k(x_hbm, idx_hbm, out_hbm, outc_hbm, idxbuf, xbuf, acc, cnt):
        cid = lax.axis_index("c")
        sid = lax.axis_index("s")
        wid = sid * NC + cid

        zero = jnp.zeros((16,), jnp.float32)

        def zero_row(r, _):
            for c in range(HID // 16):
                acc[r, pl.ds(c * 16, 16)] = zero
            return 0

        lax.fori_loop(0, NSEG + 1, zero_row, 0)
        for c in range((NSEG + 16) // 16):
            cnt[pl.ds(c * 16, 16)] = zero

        nblocks = jnp.where(wid < NBLK - (NBLK // NW) * NW, NBLK // NW + 1,
                            NBLK // NW)
        ones16 = jnp.full((16,), 1.0, jnp.float32)

        def body(j, _):
            b = wid + j * NW
            base = jnp.minimum(b * BLK, LAST_BASE)
            pltpu.sync_copy(idx_hbm.at[b], idxbuf)
            pltpu.sync_copy(x_hbm.at[pl.ds(base, BLK)], xbuf)
            pltpu.sync_copy(xbuf, acc.at[idxbuf], add=True)
            for c in range(BLK // 16):
                seg = idxbuf[pl.ds(c * 16, 16)]
                plsc.addupdate_scatter(cnt, [seg], ones16)
            return 0

        lax.fori_loop(0, nblocks, body, 0)

        pltpu.sync_copy(acc.at[pl.ds(0, NSEG)], out_hbm.at[wid])
        pltpu.sync_copy(cnt.at[pl.ds(0, NSEG)], outc_hbm.at[wid])

    return k(x, idxmat)


def _tc_finish(partials, counts, W1, b1, gamma, beta, W2, b2):
    def body(p_ref, c_ref, w1_ref, b1_ref, g_ref, be_ref, w2_ref, b2_ref,
             o_ref):
        sums = jnp.sum(p_ref[...], axis=0)
        cnt = jnp.sum(c_ref[...], axis=0)
        pooled = sums / jnp.maximum(cnt, 1.0)[:, None]
        h = jnp.dot(pooled, w1_ref[...],
                    preferred_element_type=jnp.float32,
                    precision=lax.Precision.HIGHEST) + b1_ref[...]
        mean = jnp.mean(h, axis=0, keepdims=True)
        var = jnp.mean((h - mean) * (h - mean), axis=0, keepdims=True)
        h = (h - mean) * lax.rsqrt(var + 1e-5) * g_ref[...] + be_ref[...]
        h = jnp.maximum(h, 0.0)
        o_ref[...] = jnp.dot(h, w2_ref[...],
                             preferred_element_type=jnp.float32,
                             precision=lax.Precision.HIGHEST) + b2_ref[...]

    return pl.pallas_call(
        body,
        out_shape=jax.ShapeDtypeStruct((NSEG, OUT), jnp.float32),
    )(partials, counts, W1, b1.reshape(1, HID), gamma.reshape(1, HID),
      beta.reshape(1, HID), W2, b2.reshape(1, OUT))


def kernel(x, edge_index, edge_attr, u, batch, W1, b1, gamma, beta, W2, b2):
    del edge_index, edge_attr, u
    batch_i32 = batch.astype(jnp.int32)
    head = batch_i32[: NFULL * BLK].reshape(NFULL, BLK)
    tail = jnp.concatenate(
        [jnp.full((TAIL_PAD,), NSEG, jnp.int32), batch_i32[NFULL * BLK:]]
    ).reshape(1, BLK)
    idxmat = jnp.concatenate([head, tail], axis=0)

    partials, counts = _sc_segment_sums(x, idxmat)
    return _tc_finish(partials, counts, W1, b1, gamma, beta, W2, b2)


# speedup vs baseline: 5.5090x; 5.5090x over previous
import functools

import jax
import jax.numpy as jnp
from jax import lax
from jax.experimental import pallas as pl
from jax.experimental.pallas import tpu as pltpu
from jax.experimental.pallas import tpu_sc as plsc

N_NODES = 100000
NSEG = 64
HID = 128
OUT = 128

NC = 2
NS = 16
NW = NC * NS

BLK = 128
NBLK = N_NODES // BLK
TAIL = N_NODES - NBLK * BLK
TAIL_BASE = NBLK * BLK
NB_LO = NBLK // NW
NB_REM = NBLK - NB_LO * NW


def _sc_segment_sums(x, idxmat):
    mesh = plsc.VectorSubcoreMesh(core_axis_name="c", subcore_axis_name="s")

    @functools.partial(
        pl.kernel,
        out_type=jax.ShapeDtypeStruct((NC, NSEG, HID), jnp.float32),
        mesh=mesh,
        scratch_types=[
            pltpu.VMEM((BLK,), jnp.int32),
            pltpu.VMEM((TAIL,), jnp.int32),
            pltpu.VMEM((BLK, HID), jnp.float32),
            pltpu.VMEM((8, HID), jnp.float32),
            pltpu.VMEM_SHARED((NSEG, HID), jnp.float32),
        ],
    )
    def k(x_hbm, idx_hbm, out_hbm, idxbuf, tidxbuf, xbuf, zbuf, acc):
        cid = lax.axis_index("c")
        sid = lax.axis_index("s")
        wid = sid * NC + cid

        zero = jnp.zeros((16,), jnp.float32)

        @pl.when(sid == 0)
        def _():
            for r in range(8):
                for c in range(HID // 16):
                    zbuf[r, pl.ds(c * 16, 16)] = zero
            for r in range(NSEG // 8):
                pltpu.sync_copy(zbuf, acc.at[pl.ds(r * 8, 8)])

        plsc.subcore_barrier()

        nblocks = jnp.where(wid < NB_REM, NB_LO + 1, NB_LO)

        def body(j, _):
            b = wid + j * NW
            pltpu.sync_copy(idx_hbm.at[b], idxbuf)
            pltpu.sync_copy(x_hbm.at[pl.ds(b * BLK, BLK)], xbuf)
            pltpu.sync_copy(xbuf, acc.at[idxbuf], add=True)
            return 0

        lax.fori_loop(0, nblocks, body, 0)

        @pl.when(wid == NW - 1)
        def _():
            pltpu.sync_copy(idx_hbm.at[NBLK, pl.ds(0, TAIL)], tidxbuf)
            pltpu.sync_copy(x_hbm.at[pl.ds(TAIL_BASE, TAIL)],
                            xbuf.at[pl.ds(0, TAIL)])
            pltpu.sync_copy(xbuf.at[pl.ds(0, TAIL)], acc.at[tidxbuf],
                            add=True)

        plsc.subcore_barrier()

        @pl.when(sid == 0)
        def _():
            pltpu.sync_copy(acc, out_hbm.at[cid])

    return k(x, idxmat)


def _tc_finish(partials, idxmat, W1, b1, gamma, beta, W2, b2):
    def body(p_ref, i_ref, w1_ref, b1_ref, g_ref, be_ref, w2_ref, b2_ref,
             o_ref):
        sums = jnp.sum(p_ref[...], axis=0)
        ids = i_ref[...]
        cnt = jnp.stack(
            [jnp.sum((ids == s).astype(jnp.float32)) for s in range(NSEG)]
        )
        pooled = sums / jnp.maximum(cnt, 1.0)[:, None]
        h = jnp.dot(pooled, w1_ref[...],
                    preferred_element_type=jnp.float32,
                    precision=lax.Precision.HIGHEST) + b1_ref[...]
        mean = jnp.mean(h, axis=0, keepdims=True)
        var = jnp.mean((h - mean) * (h - mean), axis=0, keepdims=True)
        h = (h - mean) * lax.rsqrt(var + 1e-5) * g_ref[...] + be_ref[...]
        h = jnp.maximum(h, 0.0)
        o_ref[...] = jnp.dot(h, w2_ref[...],
                             preferred_element_type=jnp.float32,
                             precision=lax.Precision.HIGHEST) + b2_ref[...]

    return pl.pallas_call(
        body,
        out_shape=jax.ShapeDtypeStruct((NSEG, OUT), jnp.float32),
    )(partials, idxmat, W1, b1.reshape(1, HID), gamma.reshape(1, HID),
      beta.reshape(1, HID), W2, b2.reshape(1, OUT))


def kernel(x, edge_index, edge_attr, u, batch, W1, b1, gamma, beta, W2, b2):
    del edge_index, edge_attr, u
    batch_i32 = batch.astype(jnp.int32)
    pad = (NBLK + 1) * BLK - N_NODES
    idxmat = jnp.concatenate(
        [batch_i32, jnp.full((pad,), NSEG, jnp.int32)]
    ).reshape(NBLK + 1, BLK)

    partials = _sc_segment_sums(x, idxmat)
    return _tc_finish(partials, idxmat, W1, b1, gamma, beta, W2, b2)


# speedup vs baseline: 6.4294x; 1.1671x over previous
import functools

import jax
import jax.numpy as jnp
from jax import lax
from jax.experimental import pallas as pl
from jax.experimental.pallas import tpu as pltpu
from jax.experimental.pallas import tpu_sc as plsc

N_NODES = 100000
NSEG = 64
HID = 128
OUT = 128

NC = 2
NS = 16
NW = NC * NS

BLK = 128
NBLK = N_NODES // BLK
TAIL = N_NODES - NBLK * BLK
TAIL_BASE = NBLK * BLK
NB_MIN = NBLK // NW
NB_EXTRA = NBLK - NB_MIN * NW
EXTRA_BASE = NB_MIN * NW
NCHUNK = NB_MIN // 2
CHROWS = 2 * BLK


def _sc_segment_sums(x, idxmat):
    mesh = plsc.VectorSubcoreMesh(core_axis_name="c", subcore_axis_name="s")

    @functools.partial(
        pl.kernel,
        out_type=jax.ShapeDtypeStruct((NC, NSEG, HID), jnp.float32),
        mesh=mesh,
        scratch_types=[
            pltpu.VMEM((NB_MIN, BLK), jnp.int32),
            pltpu.VMEM((BLK,), jnp.int32),
            pltpu.VMEM((TAIL,), jnp.int32),
            pltpu.VMEM((CHROWS, HID), jnp.float32),
            pltpu.VMEM((CHROWS, HID), jnp.float32),
            pltpu.VMEM((8, HID), jnp.float32),
            pltpu.VMEM_SHARED((NSEG, HID), jnp.float32),
            pltpu.SemaphoreType.DMA,
            pltpu.SemaphoreType.DMA,
        ],
    )
    def k(x_hbm, idx_hbm, out_hbm, idxall, eidxbuf, tidxbuf, xbuf0, xbuf1,
          zbuf, acc, sem0, sem1):
        cid = lax.axis_index("c")
        sid = lax.axis_index("s")
        wid = sid * NC + cid

        lo = wid * NB_MIN

        xbufs = (xbuf0, xbuf1)
        sems = (sem0, sem1)

        idx_desc = pltpu.async_copy(
            idx_hbm.at[pl.ds(lo, NB_MIN)], idxall, sem0)

        zero = jnp.zeros((16,), jnp.float32)

        @pl.when(sid == 0)
        def _():
            for r in range(8):
                for c in range(HID // 16):
                    zbuf[r, pl.ds(c * 16, 16)] = zero
            for r in range(NSEG // 8):
                pltpu.sync_copy(zbuf, acc.at[pl.ds(r * 8, 8)])

        idx_desc.wait()

        def issue(c):
            return pltpu.async_copy(
                x_hbm.at[pl.ds((lo + 2 * c) * BLK, CHROWS)],
                xbufs[c % 2], sems[c % 2])

        descs = [issue(0), None]
        plsc.subcore_barrier()

        for c in range(NCHUNK):
            if c + 1 < NCHUNK:
                descs[(c + 1) % 2] = issue(c + 1)
            descs[c % 2].wait()
            buf = xbufs[c % 2]
            pltpu.sync_copy(buf.at[pl.ds(0, BLK)],
                            acc.at[idxall.at[2 * c]], add=True)
            pltpu.sync_copy(buf.at[pl.ds(BLK, BLK)],
                            acc.at[idxall.at[2 * c + 1]], add=True)

        @pl.when(wid < NB_EXTRA)
        def _():
            b = EXTRA_BASE + wid
            pltpu.sync_copy(idx_hbm.at[b], eidxbuf)
            pltpu.sync_copy(x_hbm.at[pl.ds(b * BLK, BLK)],
                            xbuf0.at[pl.ds(0, BLK)])
            pltpu.sync_copy(xbuf0.at[pl.ds(0, BLK)],
                            acc.at[eidxbuf], add=True)

        @pl.when(wid == NW - 1)
        def _():
            pltpu.sync_copy(idx_hbm.at[NBLK, pl.ds(0, TAIL)], tidxbuf)
            pltpu.sync_copy(x_hbm.at[pl.ds(TAIL_BASE, TAIL)],
                            xbuf1.at[pl.ds(0, TAIL)])
            pltpu.sync_copy(xbuf1.at[pl.ds(0, TAIL)], acc.at[tidxbuf],
                            add=True)

        plsc.subcore_barrier()

        @pl.when(sid == 0)
        def _():
            pltpu.sync_copy(acc, out_hbm.at[cid])

    return k(x, idxmat)


def _tc_finish(partials, idxmat, W1, b1, gamma, beta, W2, b2):
    def body(p_ref, i_ref, w1_ref, b1_ref, g_ref, be_ref, w2_ref, b2_ref,
             o_ref):
        sums = jnp.sum(p_ref[...], axis=0)
        ids = i_ref[...]
        cnt = jnp.stack(
            [jnp.sum((ids == s).astype(jnp.float32)) for s in range(NSEG)]
        )
        pooled = sums / jnp.maximum(cnt, 1.0)[:, None]
        h = jnp.dot(pooled, w1_ref[...],
                    preferred_element_type=jnp.float32,
                    precision=lax.Precision.HIGHEST) + b1_ref[...]
        mean = jnp.mean(h, axis=0, keepdims=True)
        var = jnp.mean((h - mean) * (h - mean), axis=0, keepdims=True)
        h = (h - mean) * lax.rsqrt(var + 1e-5) * g_ref[...] + be_ref[...]
        h = jnp.maximum(h, 0.0)
        o_ref[...] = jnp.dot(h, w2_ref[...],
                             preferred_element_type=jnp.float32,
                             precision=lax.Precision.HIGHEST) + b2_ref[...]

    return pl.pallas_call(
        body,
        out_shape=jax.ShapeDtypeStruct((NSEG, OUT), jnp.float32),
    )(partials, idxmat, W1, b1.reshape(1, HID), gamma.reshape(1, HID),
      beta.reshape(1, HID), W2, b2.reshape(1, OUT))


def kernel(x, edge_index, edge_attr, u, batch, W1, b1, gamma, beta, W2, b2):
    del edge_index, edge_attr, u
    batch_i32 = batch.astype(jnp.int32)
    pad = (NBLK + 1) * BLK - N_NODES
    idxmat = jnp.concatenate(
        [batch_i32, jnp.full((pad,), NSEG, jnp.int32)]
    ).reshape(NBLK + 1, BLK)

    partials = _sc_segment_sums(x, idxmat)
    return _tc_finish(partials, idxmat, W1, b1, gamma, beta, W2, b2)


# speedup vs baseline: 6.4480x; 1.0029x over previous
import functools

import jax
import jax.numpy as jnp
from jax import lax
from jax.experimental import pallas as pl
from jax.experimental.pallas import tpu as pltpu
from jax.experimental.pallas import tpu_sc as plsc

N_NODES = 100000
NSEG = 64
HID = 128
OUT = 128

NC = 2
NS = 16
NW = NC * NS

BLK = 128
NBLK = N_NODES // BLK
TAIL = N_NODES - NBLK * BLK
TAIL_BASE = NBLK * BLK
NB_MIN = NBLK // NW
NB_EXTRA = NBLK - NB_MIN * NW
EXTRA_BASE = NB_MIN * NW
NCHUNK = NB_MIN // 2
CHROWS = 2 * BLK


def _sc_segment_sums(x, idxmat):
    mesh = plsc.VectorSubcoreMesh(core_axis_name="c", subcore_axis_name="s")

    @functools.partial(
        pl.kernel,
        out_type=jax.ShapeDtypeStruct((NC, NSEG, HID), jnp.float32),
        mesh=mesh,
        scratch_types=[
            pltpu.VMEM((NB_MIN, BLK), jnp.int32),
            pltpu.VMEM((BLK,), jnp.int32),
            pltpu.VMEM((TAIL,), jnp.int32),
            pltpu.VMEM((CHROWS, HID), jnp.float32),
            pltpu.VMEM((CHROWS, HID), jnp.float32),
            pltpu.VMEM((CHROWS, HID), jnp.float32),
            pltpu.VMEM((8, HID), jnp.float32),
            pltpu.VMEM_SHARED((NSEG, HID), jnp.float32),
            pltpu.SemaphoreType.DMA,
            pltpu.SemaphoreType.DMA,
            pltpu.SemaphoreType.DMA,
            pltpu.SemaphoreType.DMA,
            pltpu.SemaphoreType.DMA,
            pltpu.SemaphoreType.DMA,
        ],
    )
    def k(x_hbm, idx_hbm, out_hbm, idxall, eidxbuf, tidxbuf, xbuf0, xbuf1,
          xbuf2, zbuf, acc, sem0, sem1, sem2, ssem0, ssem1, ssem2):
        cid = lax.axis_index("c")
        sid = lax.axis_index("s")
        wid = sid * NC + cid

        lo = wid * NB_MIN

        xbufs = (xbuf0, xbuf1, xbuf2)
        sems = (sem0, sem1, sem2)
        ssems = (ssem0, ssem1, ssem2)
        NBUF = 3

        idx_desc = pltpu.async_copy(
            idx_hbm.at[pl.ds(lo, NB_MIN)], idxall, sem0)

        zero = jnp.zeros((16,), jnp.float32)

        @pl.when(sid == 0)
        def _():
            for r in range(8):
                for c in range(HID // 16):
                    zbuf[r, pl.ds(c * 16, 16)] = zero
            for r in range(NSEG // 8):
                pltpu.sync_copy(zbuf, acc.at[pl.ds(r * 8, 8)])

        idx_desc.wait()

        def issue(c):
            return pltpu.async_copy(
                x_hbm.at[pl.ds((lo + 2 * c) * BLK, CHROWS)],
                xbufs[c % NBUF], sems[c % NBUF])

        dma = [issue(0), issue(1), None]
        sct = [None, None, None]
        plsc.subcore_barrier()

        for c in range(NCHUNK):
            s = c % NBUF
            dma[s].wait()
            buf = xbufs[s]
            d0 = pltpu.make_async_copy(buf.at[pl.ds(0, BLK)],
                                       acc.at[idxall.at[2 * c]], ssems[s])
            d1 = pltpu.make_async_copy(buf.at[pl.ds(BLK, BLK)],
                                       acc.at[idxall.at[2 * c + 1]],
                                       ssems[s])
            d0.start(add=True)
            d1.start(add=True)
            sct[s] = (d0, d1)
            if c + 2 < NCHUNK:
                t = (c + 2) % NBUF
                if sct[t] is not None:
                    sct[t][0].wait()
                    sct[t][1].wait()
                    sct[t] = None
                dma[t] = issue(c + 2)

        for s in range(NBUF):
            if sct[s] is not None:
                sct[s][0].wait()
                sct[s][1].wait()

        @pl.when(wid < NB_EXTRA)
        def _():
            b = EXTRA_BASE + wid
            pltpu.sync_copy(idx_hbm.at[b], eidxbuf)
            pltpu.sync_copy(x_hbm.at[pl.ds(b * BLK, BLK)],
                            xbuf0.at[pl.ds(0, BLK)])
            pltpu.sync_copy(xbuf0.at[pl.ds(0, BLK)],
                            acc.at[eidxbuf], add=True)

        @pl.when(wid == NW - 1)
        def _():
            pltpu.sync_copy(idx_hbm.at[NBLK, pl.ds(0, TAIL)], tidxbuf)
            pltpu.sync_copy(x_hbm.at[pl.ds(TAIL_BASE, TAIL)],
                            xbuf1.at[pl.ds(0, TAIL)])
            pltpu.sync_copy(xbuf1.at[pl.ds(0, TAIL)], acc.at[tidxbuf],
                            add=True)

        plsc.subcore_barrier()

        @pl.when(sid == 0)
        def _():
            pltpu.sync_copy(acc, out_hbm.at[cid])

    return k(x, idxmat)


def _tc_finish(partials, idxmat, W1, b1, gamma, beta, W2, b2):
    def body(p_ref, i_ref, w1_ref, b1_ref, g_ref, be_ref, w2_ref, b2_ref,
             o_ref):
        sums = jnp.sum(p_ref[...], axis=0)
        ids = i_ref[...]
        cnt = jnp.stack(
            [jnp.sum((ids == s).astype(jnp.float32)) for s in range(NSEG)]
        )
        pooled = sums / jnp.maximum(cnt, 1.0)[:, None]
        h = jnp.dot(pooled, w1_ref[...],
                    preferred_element_type=jnp.float32,
                    precision=lax.Precision.HIGHEST) + b1_ref[...]
        mean = jnp.mean(h, axis=0, keepdims=True)
        var = jnp.mean((h - mean) * (h - mean), axis=0, keepdims=True)
        h = (h - mean) * lax.rsqrt(var + 1e-5) * g_ref[...] + be_ref[...]
        h = jnp.maximum(h, 0.0)
        o_ref[...] = jnp.dot(h, w2_ref[...],
                             preferred_element_type=jnp.float32,
                             precision=lax.Precision.HIGHEST) + b2_ref[...]

    return pl.pallas_call(
        body,
        out_shape=jax.ShapeDtypeStruct((NSEG, OUT), jnp.float32),
    )(partials, idxmat, W1, b1.reshape(1, HID), gamma.reshape(1, HID),
      beta.reshape(1, HID), W2, b2.reshape(1, OUT))


def kernel(x, edge_index, edge_attr, u, batch, W1, b1, gamma, beta, W2, b2):
    del edge_index, edge_attr, u
    batch_i32 = batch.astype(jnp.int32)
    pad = (NBLK + 1) * BLK - N_NODES
    idxmat = jnp.concatenate(
        [batch_i32, jnp.full((pad,), NSEG, jnp.int32)]
    ).reshape(NBLK + 1, BLK)

    partials = _sc_segment_sums(x, idxmat)
    return _tc_finish(partials, idxmat, W1, b1, gamma, beta, W2, b2)
